# baseline (speedup 1.0000x reference)
import jax
import jax.numpy as jnp
from jax.experimental import pallas as pl
from jax.experimental.pallas import tpu as pltpu

DIM = 256
TOTAL = 32768
NSEG = 16
R = 2048
NSTREAM = 2
HALF = TOTAL // NSTREAM
NBLK = HALF // R
SUB = 128


def _readout_body(batch_ref, xa_ref, xb_ref, w1t_ref, b1_ref, w2_ref,
                  w2s_ref, b2_ref, out_ref,
                  cosa_ref, sina_ref, starts_ref):
    pid = pl.program_id(0)

    lane = jax.lax.broadcasted_iota(jnp.int32, (1, DIM), 1)
    odd = (lane % 2) == 1
    theta = jnp.exp((lane - (lane % 2)).astype(jnp.float32) *
                    (-jnp.log(10000.0) / DIM))

    @pl.when(pid == 0)
    def _prologue():
        bt = batch_ref[...]
        for s in range(NSEG):
            starts_ref[s] = jnp.sum((bt < s).astype(jnp.int32))
        starts_ref[NSEG] = jnp.int32(TOTAL)
        starts_ref[NSEG + 1] = jnp.max(bt)
        u = jax.lax.broadcasted_iota(jnp.int32, (SUB, 1), 0).astype(jnp.float32)
        au = u * theta
        cu = jnp.cos(au)
        su = jnp.sin(au)
        qv = jax.lax.broadcasted_iota(jnp.int32, (R // SUB, 1), 0)
        aq = (qv * SUB).astype(jnp.float32) * theta
        cq = jnp.cos(aq)
        sq = jnp.sin(aq)
        for q in range(R // SUB):
            cqr = cq[q:q + 1, :]
            sqr = sq[q:q + 1, :]
            ca = cu * cqr - su * sqr
            sa = su * cqr + cu * sqr
            cosa_ref[q * SUB:(q + 1) * SUB, :] = ca
            sina_ref[q * SUB:(q + 1) * SUB, :] = jnp.where(odd, -sa, sa)

    last_id = starts_ref[NSEG + 1]
    w2 = w2_ref[...]
    w2s = w2s_ref[...]
    cosa = cosa_ref[...]
    sina = sina_ref[...]
    w1t = w1t_ref[...]
    b1 = b1_ref[...]
    b2 = b2_ref[0, 0]

    lane16 = jax.lax.broadcasted_iota(jnp.int32, (1, NSEG), 1)
    seg = jax.lax.broadcasted_iota(jnp.int32, (NSEG, 1), 0)
    starts_row = jnp.zeros((1, NSEG), jnp.int32)
    next_row = jnp.zeros((1, NSEG), jnp.int32)
    starts_col = jnp.zeros((NSEG, 1), jnp.int32)
    for s in range(NSEG):
        starts_row = jnp.where(lane16 == s, starts_ref[s], starts_row)
        next_row = jnp.where(lane16 == s, starts_ref[s + 1], next_row)
        starts_col = jnp.where(seg == s, starts_ref[s], starts_col)
    is_last = seg == last_id

    rloc = jax.lax.broadcasted_iota(jnp.int32, (R, 1), 0)

    for k in range(NSTREAM):
        base = k * HALF + pid * R
        offb = (base - starts_col).astype(jnp.float32) * theta
        cb = jnp.cos(offb)
        sb = jnp.sin(offb)
        sgn_sb = jnp.where(odd, -sb, sb)
        p_tab = cb * w2 + sgn_sb * w2s
        q_tab = cb * w2s - sgn_sb * w2
        p_tab = jnp.where(is_last, 0.0, p_tab)
        q_tab = jnp.where(is_last, 0.0, q_tab)
        c_tab = jnp.where(is_last, w2, 0.0)
        tab = jnp.concatenate([p_tab, q_tab, c_tab], axis=1)

        row = rloc + base
        ind = ((row >= starts_row) & (row < next_row)).astype(jnp.float32)
        sel = jnp.dot(ind, tab, preferred_element_type=jnp.float32)
        coef = (cosa * sel[:, :DIM] + sina * sel[:, DIM:2 * DIM] +
                sel[:, 2 * DIM:])
        xr = xa_ref if k == 0 else xb_ref
        h = jnp.dot(xr[0], w1t, preferred_element_type=jnp.float32) + b1
        out_ref[k] = jnp.sum(h * coef, axis=1, keepdims=True) + b2


def kernel(x, batch, W1, b1, W2, b2):
    w1t = W1.T
    b1r = b1.reshape(1, DIM)
    w2 = W2.reshape(1, DIM)
    w2s = W2.reshape(DIM // 2, 2)[:, ::-1].reshape(1, DIM)
    b2r = b2.reshape(1, 1)
    bt = batch.reshape(TOTAL // 128, 128)
    x3 = x.reshape(NSTREAM, HALF, DIM)

    out = pl.pallas_call(
        _readout_body,
        grid=(NBLK,),
        in_specs=[
            pl.BlockSpec((TOTAL // 128, 128), lambda i: (0, 0)),
            pl.BlockSpec((1, R, DIM), lambda i: (0, i, 0)),
            pl.BlockSpec((1, R, DIM), lambda i: (1, i, 0)),
            pl.BlockSpec((DIM, DIM), lambda i: (0, 0)),
            pl.BlockSpec((1, DIM), lambda i: (0, 0)),
            pl.BlockSpec((1, DIM), lambda i: (0, 0)),
            pl.BlockSpec((1, DIM), lambda i: (0, 0)),
            pl.BlockSpec((1, 1), lambda i: (0, 0)),
        ],
        out_specs=pl.BlockSpec((NSTREAM, R, 1), lambda i: (0, i, 0)),
        out_shape=jax.ShapeDtypeStruct((NSTREAM, HALF, 1), jnp.float32),
        scratch_shapes=[
            pltpu.VMEM((R, DIM), jnp.float32),
            pltpu.VMEM((R, DIM), jnp.float32),
            pltpu.SMEM((NSEG + 2,), jnp.int32),
        ],
        compiler_params=pltpu.CompilerParams(
            dimension_semantics=("arbitrary",),
        ),
    )(bt, x3, x3, w1t, b1r, w2, w2s, b2r)
    return out.reshape(TOTAL, 1)


# speedup vs baseline: 1.0026x; 1.0026x over previous
import jax
import jax.numpy as jnp
from jax.experimental import pallas as pl
from jax.experimental.pallas import tpu as pltpu

DIM = 256
TOTAL = 32768
NSEG = 16
R = 2048
NSTREAM = 2
HALF = TOTAL // NSTREAM
NBLK = HALF // R
SUB = 128


def _start_copies(x_ref, xbuf_ref, sems_ref, step, slot):
    for k in range(NSTREAM):
        pltpu.make_async_copy(
            x_ref.at[pl.ds(k * HALF + step * R, R), :],
            xbuf_ref.at[k, slot],
            sems_ref.at[k, slot],
        ).start()


def _wait_copies(x_ref, xbuf_ref, sems_ref, step, slot):
    for k in range(NSTREAM):
        pltpu.make_async_copy(
            x_ref.at[pl.ds(k * HALF + step * R, R), :],
            xbuf_ref.at[k, slot],
            sems_ref.at[k, slot],
        ).wait()


def _readout_body(batch_ref, x_ref, w1t_ref, b1_ref, w2_ref,
                  w2s_ref, b2_ref, out_ref,
                  cosa_ref, sina_ref, starts_ref, xbuf_ref, sems_ref):
    pid = pl.program_id(0)
    slot = jax.lax.rem(pid, 2)
    nxt = jax.lax.rem(pid + 1, 2)

    lane = jax.lax.broadcasted_iota(jnp.int32, (1, DIM), 1)
    odd = (lane % 2) == 1
    theta = jnp.exp((lane - (lane % 2)).astype(jnp.float32) *
                    (-jnp.log(10000.0) / DIM))

    @pl.when(pid == 0)
    def _first_fetch():
        _start_copies(x_ref, xbuf_ref, sems_ref, 0, 0)

    @pl.when(pid + 1 < NBLK)
    def _prefetch():
        _start_copies(x_ref, xbuf_ref, sems_ref, pid + 1, nxt)

    @pl.when(pid == 0)
    def _prologue():
        bt = batch_ref[...]
        for s in range(NSEG):
            starts_ref[s] = jnp.sum((bt < s).astype(jnp.int32))
        starts_ref[NSEG] = jnp.int32(TOTAL)
        starts_ref[NSEG + 1] = jnp.max(bt)
        u = jax.lax.broadcasted_iota(jnp.int32, (SUB, 1), 0).astype(jnp.float32)
        au = u * theta
        cu = jnp.cos(au)
        su = jnp.sin(au)
        qv = jax.lax.broadcasted_iota(jnp.int32, (R // SUB, 1), 0)
        aq = (qv * SUB).astype(jnp.float32) * theta
        cq = jnp.cos(aq)
        sq = jnp.sin(aq)
        for q in range(R // SUB):
            cqr = cq[q:q + 1, :]
            sqr = sq[q:q + 1, :]
            ca = cu * cqr - su * sqr
            sa = su * cqr + cu * sqr
            cosa_ref[q * SUB:(q + 1) * SUB, :] = ca
            sina_ref[q * SUB:(q + 1) * SUB, :] = jnp.where(odd, -sa, sa)

    last_id = starts_ref[NSEG + 1]
    w2 = w2_ref[...]
    w2s = w2s_ref[...]
    cosa = cosa_ref[...]
    sina = sina_ref[...]
    w1t = w1t_ref[...]
    b1 = b1_ref[...]
    b2 = b2_ref[0, 0]

    lane16 = jax.lax.broadcasted_iota(jnp.int32, (1, NSEG), 1)
    seg = jax.lax.broadcasted_iota(jnp.int32, (NSEG, 1), 0)
    starts_row = jnp.zeros((1, NSEG), jnp.int32)
    next_row = jnp.zeros((1, NSEG), jnp.int32)
    starts_col = jnp.zeros((NSEG, 1), jnp.int32)
    for s in range(NSEG):
        starts_row = jnp.where(lane16 == s, starts_ref[s], starts_row)
        next_row = jnp.where(lane16 == s, starts_ref[s + 1], next_row)
        starts_col = jnp.where(seg == s, starts_ref[s], starts_col)
    is_last = seg == last_id

    rloc = jax.lax.broadcasted_iota(jnp.int32, (R, 1), 0)

    _wait_copies(x_ref, xbuf_ref, sems_ref, pid, slot)

    for k in range(NSTREAM):
        base = k * HALF + pid * R
        offb = (base - starts_col).astype(jnp.float32) * theta
        cb = jnp.cos(offb)
        sb = jnp.sin(offb)
        sgn_sb = jnp.where(odd, -sb, sb)
        p_tab = cb * w2 + sgn_sb * w2s
        q_tab = cb * w2s - sgn_sb * w2
        p_tab = jnp.where(is_last, 0.0, p_tab)
        q_tab = jnp.where(is_last, 0.0, q_tab)
        c_tab = jnp.where(is_last, w2, 0.0)
        tab = jnp.concatenate([p_tab, q_tab, c_tab], axis=1)

        row = rloc + base
        ind = ((row >= starts_row) & (row < next_row)).astype(jnp.float32)
        sel = jnp.dot(ind, tab, preferred_element_type=jnp.float32)
        coef = (cosa * sel[:, :DIM] + sina * sel[:, DIM:2 * DIM] +
                sel[:, 2 * DIM:])
        h = jnp.dot(xbuf_ref[k, slot], w1t,
                    preferred_element_type=jnp.float32) + b1
        out_ref[k] = jnp.sum(h * coef, axis=1, keepdims=True) + b2


def kernel(x, batch, W1, b1, W2, b2):
    w1t = W1.T
    b1r = b1.reshape(1, DIM)
    w2 = W2.reshape(1, DIM)
    w2s = W2.reshape(DIM // 2, 2)[:, ::-1].reshape(1, DIM)
    b2r = b2.reshape(1, 1)
    bt = batch.reshape(TOTAL // 128, 128)

    out = pl.pallas_call(
        _readout_body,
        grid=(NBLK,),
        in_specs=[
            pl.BlockSpec((TOTAL // 128, 128), lambda i: (0, 0)),
            pl.BlockSpec(memory_space=pl.ANY),
            pl.BlockSpec((DIM, DIM), lambda i: (0, 0)),
            pl.BlockSpec((1, DIM), lambda i: (0, 0)),
            pl.BlockSpec((1, DIM), lambda i: (0, 0)),
            pl.BlockSpec((1, DIM), lambda i: (0, 0)),
            pl.BlockSpec((1, 1), lambda i: (0, 0)),
        ],
        out_specs=pl.BlockSpec((NSTREAM, R, 1), lambda i: (0, i, 0)),
        out_shape=jax.ShapeDtypeStruct((NSTREAM, HALF, 1), jnp.float32),
        scratch_shapes=[
            pltpu.VMEM((R, DIM), jnp.float32),
            pltpu.VMEM((R, DIM), jnp.float32),
            pltpu.SMEM((NSEG + 2,), jnp.int32),
            pltpu.VMEM((NSTREAM, 2, R, DIM), jnp.float32),
            pltpu.SemaphoreType.DMA((NSTREAM, 2)),
        ],
        compiler_params=pltpu.CompilerParams(
            dimension_semantics=("arbitrary",),
        ),
    )(bt, x, w1t, b1r, w2, w2s, b2r)
    return out.reshape(TOTAL, 1)


# speedup vs baseline: 1.0101x; 1.0075x over previous
import jax
import jax.numpy as jnp
from jax.experimental import pallas as pl
from jax.experimental.pallas import tpu as pltpu

DIM = 256
TOTAL = 32768
NSEG = 16
R = 2048
NSTREAM = 2
HALF = TOTAL // NSTREAM
NBLK = HALF // R
SUB = 128


def _start_copies(x_ref, xbuf_ref, sems_ref, step, slot):
    for k in range(NSTREAM):
        pltpu.make_async_copy(
            x_ref.at[pl.ds(k * HALF + step * R, R), :],
            xbuf_ref.at[k, slot],
            sems_ref.at[k, slot],
        ).start()


def _wait_copies(x_ref, xbuf_ref, sems_ref, step, slot):
    for k in range(NSTREAM):
        pltpu.make_async_copy(
            x_ref.at[pl.ds(k * HALF + step * R, R), :],
            xbuf_ref.at[k, slot],
            sems_ref.at[k, slot],
        ).wait()


def _readout_body(batch_ref, x_ref, w1t_ref, b1_ref, w2_ref,
                  w2s_ref, b2_ref, out_ref,
                  cosa_ref, sina_ref, starts_ref, xbuf_ref, sems_ref):
    pid = pl.program_id(0)
    slot = jax.lax.rem(pid, 2)
    nxt = jax.lax.rem(pid + 1, 2)

    lane = jax.lax.broadcasted_iota(jnp.int32, (1, DIM), 1)
    odd = (lane % 2) == 1
    theta = jnp.exp((lane - (lane % 2)).astype(jnp.float32) *
                    (-jnp.log(10000.0) / DIM))

    @pl.when(pid == 0)
    def _first_fetch():
        _start_copies(x_ref, xbuf_ref, sems_ref, 0, 0)

    @pl.when(pid + 1 < NBLK)
    def _prefetch():
        _start_copies(x_ref, xbuf_ref, sems_ref, pid + 1, nxt)

    @pl.when(pid == 0)
    def _prologue():
        bt = batch_ref[...]
        for s in range(NSEG):
            starts_ref[s] = jnp.sum((bt < s).astype(jnp.int32))
        starts_ref[NSEG] = jnp.int32(TOTAL)
        starts_ref[NSEG + 1] = jnp.max(bt)
        u = jax.lax.broadcasted_iota(jnp.int32, (SUB, 1), 0).astype(jnp.float32)
        au = u * theta
        cu = jnp.cos(au)
        su = jnp.sin(au)
        qv = jax.lax.broadcasted_iota(jnp.int32, (R // SUB, 1), 0)
        aq = (qv * SUB).astype(jnp.float32) * theta
        cq = jnp.cos(aq)
        sq = jnp.sin(aq)
        for q in range(R // SUB):
            cqr = cq[q:q + 1, :]
            sqr = sq[q:q + 1, :]
            ca = cu * cqr - su * sqr
            sa = su * cqr + cu * sqr
            cosa_ref[q * SUB:(q + 1) * SUB, :] = ca.astype(jnp.bfloat16)
            sina_ref[q * SUB:(q + 1) * SUB, :] = (
                jnp.where(odd, -sa, sa).astype(jnp.bfloat16))

    last_id = starts_ref[NSEG + 1]
    w2 = w2_ref[...]
    w2s = w2s_ref[...]
    cosa = cosa_ref[...]
    sina = sina_ref[...]
    w1t = w1t_ref[...]
    b1 = b1_ref[...].astype(jnp.bfloat16)
    b2 = b2_ref[0, 0]
    w2c = w2.astype(jnp.bfloat16).reshape(DIM, 1)

    lane16 = jax.lax.broadcasted_iota(jnp.int32, (1, NSEG), 1)
    seg = jax.lax.broadcasted_iota(jnp.int32, (NSEG, 1), 0)
    starts_row = jnp.zeros((1, NSEG), jnp.int32)
    next_row = jnp.zeros((1, NSEG), jnp.int32)
    starts_col = jnp.zeros((NSEG, 1), jnp.int32)
    for s in range(NSEG):
        starts_row = jnp.where(lane16 == s, starts_ref[s], starts_row)
        next_row = jnp.where(lane16 == s, starts_ref[s + 1], next_row)
        starts_col = jnp.where(seg == s, starts_ref[s], starts_col)
    is_last = seg == last_id

    rloc = jax.lax.broadcasted_iota(jnp.int32, (R, 1), 0)

    _wait_copies(x_ref, xbuf_ref, sems_ref, pid, slot)

    rot_end = jnp.int32(0)
    for s in range(NSEG):
        rot_end = jnp.where(last_id == s, starts_ref[s], rot_end)

    for k in range(NSTREAM):
        base = k * HALF + pid * R
        offb = (base - starts_col).astype(jnp.float32) * theta
        cb = jnp.cos(offb)
        sb = jnp.sin(offb)
        sgn_sb = jnp.where(odd, -sb, sb)
        p_tab = cb * w2 + sgn_sb * w2s
        q_tab = cb * w2s - sgn_sb * w2
        tab = jnp.concatenate([p_tab, q_tab], axis=1).astype(jnp.bfloat16)

        row = rloc + base
        ind = ((row >= starts_row) & (row < next_row)).astype(jnp.bfloat16)
        sel = jnp.dot(ind, tab,
                      preferred_element_type=jnp.float32).astype(jnp.bfloat16)
        h = jnp.dot(xbuf_ref[k, slot], w1t,
                    preferred_element_type=jnp.float32).astype(jnp.bfloat16) + b1
        coef = cosa * sel[:, :DIM] + sina * sel[:, DIM:]
        hc = (h * coef).astype(jnp.float32)
        out_rot = jnp.sum(hc, axis=1, keepdims=True)
        out_plain = jnp.dot(h, w2c, preferred_element_type=jnp.float32)
        out_ref[k] = jnp.where(row < rot_end, out_rot, out_plain) + b2


def kernel(x, batch, W1, b1, W2, b2):
    w1t = W1.T
    b1r = b1.reshape(1, DIM)
    w2 = W2.reshape(1, DIM)
    w2s = W2.reshape(DIM // 2, 2)[:, ::-1].reshape(1, DIM)
    b2r = b2.reshape(1, 1)
    bt = batch.reshape(TOTAL // 128, 128)

    out = pl.pallas_call(
        _readout_body,
        grid=(NBLK,),
        in_specs=[
            pl.BlockSpec((TOTAL // 128, 128), lambda i: (0, 0)),
            pl.BlockSpec(memory_space=pl.ANY),
            pl.BlockSpec((DIM, DIM), lambda i: (0, 0)),
            pl.BlockSpec((1, DIM), lambda i: (0, 0)),
            pl.BlockSpec((1, DIM), lambda i: (0, 0)),
            pl.BlockSpec((1, DIM), lambda i: (0, 0)),
            pl.BlockSpec((1, 1), lambda i: (0, 0)),
        ],
        out_specs=pl.BlockSpec((NSTREAM, R, 1), lambda i: (0, i, 0)),
        out_shape=jax.ShapeDtypeStruct((NSTREAM, HALF, 1), jnp.float32),
        scratch_shapes=[
            pltpu.VMEM((R, DIM), jnp.bfloat16),
            pltpu.VMEM((R, DIM), jnp.bfloat16),
            pltpu.SMEM((NSEG + 2,), jnp.int32),
            pltpu.VMEM((NSTREAM, 2, R, DIM), jnp.float32),
            pltpu.SemaphoreType.DMA((NSTREAM, 2)),
        ],
        compiler_params=pltpu.CompilerParams(
            dimension_semantics=("arbitrary",),
        ),
    )(bt, x, w1t, b1r, w2, w2s, b2r)
    return out.reshape(TOTAL, 1)


# speedup vs baseline: 1.0177x; 1.0075x over previous
import jax
import jax.numpy as jnp
from jax.experimental import pallas as pl
from jax.experimental.pallas import tpu as pltpu

DIM = 256
TOTAL = 32768
NSEG = 16
RB = 8192
R = 2048
NCHUNK = RB // R
NBLK = TOTAL // RB
SUB = 128


def _readout_body(batch_ref, x_ref, w1t_ref, b1_ref, w2_ref,
                  w2s_ref, b2_ref, out_ref,
                  cosa_ref, sina_ref, starts_ref):
    pid = pl.program_id(0)

    lane = jax.lax.broadcasted_iota(jnp.int32, (1, DIM), 1)
    odd = (lane % 2) == 1
    theta = jnp.exp((lane - (lane % 2)).astype(jnp.float32) *
                    (-jnp.log(10000.0) / DIM))

    @pl.when(pid == 0)
    def _prologue():
        bt = batch_ref[...]
        for s in range(NSEG):
            starts_ref[s] = jnp.sum((bt < s).astype(jnp.int32))
        starts_ref[NSEG] = jnp.int32(TOTAL)
        starts_ref[NSEG + 1] = jnp.max(bt)
        u = jax.lax.broadcasted_iota(jnp.int32, (SUB, 1), 0).astype(jnp.float32)
        au = u * theta
        cu = jnp.cos(au)
        su = jnp.sin(au)
        qv = jax.lax.broadcasted_iota(jnp.int32, (R // SUB, 1), 0)
        aq = (qv * SUB).astype(jnp.float32) * theta
        cq = jnp.cos(aq)
        sq = jnp.sin(aq)
        for q in range(R // SUB):
            cqr = cq[q:q + 1, :]
            sqr = sq[q:q + 1, :]
            ca = cu * cqr - su * sqr
            sa = su * cqr + cu * sqr
            cosa_ref[q * SUB:(q + 1) * SUB, :] = ca
            sina_ref[q * SUB:(q + 1) * SUB, :] = jnp.where(odd, -sa, sa)

    last_id = starts_ref[NSEG + 1]
    w2 = w2_ref[...]
    w2s = w2s_ref[...]
    cosa = cosa_ref[...]
    sina = sina_ref[...]
    w1t = w1t_ref[...]
    b1 = b1_ref[...]
    b2 = b2_ref[0, 0]

    lane16 = jax.lax.broadcasted_iota(jnp.int32, (1, NSEG), 1)
    seg = jax.lax.broadcasted_iota(jnp.int32, (NSEG, 1), 0)
    starts_row = jnp.zeros((1, NSEG), jnp.int32)
    next_row = jnp.zeros((1, NSEG), jnp.int32)
    starts_col = jnp.zeros((NSEG, 1), jnp.int32)
    for s in range(NSEG):
        starts_row = jnp.where(lane16 == s, starts_ref[s], starts_row)
        next_row = jnp.where(lane16 == s, starts_ref[s + 1], next_row)
        starts_col = jnp.where(seg == s, starts_ref[s], starts_col)
    is_last = seg == last_id

    rloc = jax.lax.broadcasted_iota(jnp.int32, (R, 1), 0)

    for c in range(NCHUNK):
        base = pid * RB + c * R
        offb = (base - starts_col).astype(jnp.float32) * theta
        cb = jnp.cos(offb)
        sb = jnp.sin(offb)
        sgn_sb = jnp.where(odd, -sb, sb)
        p_tab = cb * w2 + sgn_sb * w2s
        q_tab = cb * w2s - sgn_sb * w2
        p_tab = jnp.where(is_last, 0.0, p_tab)
        q_tab = jnp.where(is_last, 0.0, q_tab)
        c_tab = jnp.where(is_last, w2, 0.0)
        tab = jnp.concatenate([p_tab, q_tab, c_tab], axis=1)

        row = rloc + base
        ind = ((row >= starts_row) & (row < next_row)).astype(jnp.float32)
        sel = jnp.dot(ind, tab, preferred_element_type=jnp.float32)
        coef = (cosa * sel[:, :DIM] + sina * sel[:, DIM:2 * DIM] +
                sel[:, 2 * DIM:])
        h = jnp.dot(x_ref[c * R:(c + 1) * R, :], w1t,
                    preferred_element_type=jnp.float32) + b1
        out_ref[c * R:(c + 1) * R, :] = (
            jnp.sum(h * coef, axis=1, keepdims=True) + b2)


def kernel(x, batch, W1, b1, W2, b2):
    w1t = W1.T
    b1r = b1.reshape(1, DIM)
    w2 = W2.reshape(1, DIM)
    w2s = W2.reshape(DIM // 2, 2)[:, ::-1].reshape(1, DIM)
    b2r = b2.reshape(1, 1)
    bt = batch.reshape(TOTAL // 128, 128)

    out = pl.pallas_call(
        _readout_body,
        grid=(NBLK,),
        in_specs=[
            pl.BlockSpec((TOTAL // 128, 128), lambda i: (0, 0)),
            pl.BlockSpec((RB, DIM), lambda i: (i, 0)),
            pl.BlockSpec((DIM, DIM), lambda i: (0, 0)),
            pl.BlockSpec((1, DIM), lambda i: (0, 0)),
            pl.BlockSpec((1, DIM), lambda i: (0, 0)),
            pl.BlockSpec((1, DIM), lambda i: (0, 0)),
            pl.BlockSpec((1, 1), lambda i: (0, 0)),
        ],
        out_specs=pl.BlockSpec((RB, 1), lambda i: (i, 0)),
        out_shape=jax.ShapeDtypeStruct((TOTAL, 1), jnp.float32),
        scratch_shapes=[
            pltpu.VMEM((R, DIM), jnp.float32),
            pltpu.VMEM((R, DIM), jnp.float32),
            pltpu.SMEM((NSEG + 2,), jnp.int32),
        ],
        compiler_params=pltpu.CompilerParams(
            dimension_semantics=("arbitrary",),
        ),
    )(bt, x, w1t, b1r, w2, w2s, b2r)
    return out
